# Initial kernel scaffold; baseline (speedup 1.0000x reference)
import jax
import jax.numpy as jnp
from jax.experimental import pallas as pl


def kernel(net, inp, corr, flow, ii, jj, kk, kk_idx_map, G_kk, ij_idx_map, G_ij, params):
    raise NotImplementedError("write your pallas kernel here")


# speedup vs baseline: 3.2545x; 3.2545x over previous
import functools

import jax
import jax.numpy as jnp
from jax.experimental import pallas as pl

DIM = 384
N = 4096
CORR_DIM = 882
G_KK_C = 512
G_IJ_C = 64
BLK = 256
NBLK = N // BLK

f32 = jnp.float32


def _dgT(x, w):
    return jax.lax.dot_general(
        x, w, dimension_numbers=(((1,), (1,)), ((), ())),
        preferred_element_type=f32)


def _dg(x, w):
    return jax.lax.dot_general(
        x, w, dimension_numbers=(((1,), (0,)), ((), ())),
        preferred_element_type=f32)


def _dgTT(x, w):
    return jax.lax.dot_general(
        x, w, dimension_numbers=(((0,), (0,)), ((), ())),
        preferred_element_type=f32)


def _ln(x, g, b, eps=1e-3):
    mu = jnp.mean(x, axis=-1, keepdims=True)
    var = jnp.mean((x - mu) ** 2, axis=-1, keepdims=True)
    return (x - mu) / jnp.sqrt(var + eps) * g + b


def _k1(corr_ref, net_ref, inp_ref, ii_ref,
        w1, b1, w2, b2, lng, lnb, w3, b3, ng, nb, out_ref):
    c = jax.nn.relu(_dgT(corr_ref[...], w1[...]) + b1[...])
    c = _dgT(c, w2[...]) + b2[...]
    c = _ln(c, lng[...], lnb[...])
    c = jax.nn.relu(c)
    c = _dgT(c, w3[...]) + b3[...]
    ii_bias = jnp.sum(ii_ref[...]) * 1e-10
    x = net_ref[...] + inp_ref[...] + c + ii_bias
    out_ref[...] = _ln(x, ng[...], nb[...])


def _k2a(net_ref, kk_row_ref, kk_col_ref, jj_row_ref, jj_col_ref,
         c1w1, c1b1, c1w2, c1b2, out_ref, jx_ref):
    net_a = net_ref[...]
    kk_row = kk_row_ref[...]
    kk_col = kk_col_ref[...]
    jj_row = jj_row_ref[...]
    jj_col = jj_col_ref[...]

    iota = jax.lax.broadcasted_iota(jnp.int32, (BLK, N), 1)
    jj_b = jnp.broadcast_to(jj_row, (BLK, N))

    for b in range(NBLK):
        sl = slice(b * BLK, (b + 1) * BLK)
        kc = kk_col[sl]
        jc = jj_col[sl]
        mask = kk_row == kc
        prev = jnp.where(mask & (jj_row < jc), jj_b, 0)
        m = jnp.max(prev, axis=1, keepdims=True)
        ixb = jnp.min(jnp.where(prev == m, iota, N), axis=1, keepdims=True)
        nxt = jnp.where(mask & (jj_row > jc), jj_b, N)
        mn = jnp.min(nxt, axis=1, keepdims=True)
        jx_ref[sl] = jnp.min(jnp.where(nxt == mn, iota, N), axis=1,
                             keepdims=True)
        oh = (iota == ixb).astype(f32)
        gath = _dg(oh, net_a)
        h = jax.nn.relu(_dgT(gath, c1w1[...]) + c1b1[...])
        upd = _dgT(h, c1w2[...]) + c1b2[...]
        out_ref[sl] = net_a[sl] + upd


def _k2b(net_ref, jx_ref, c2w1, c2b1, c2w2, c2b2, out_ref):
    net_b = net_ref[...]
    jx = jx_ref[...]
    iota = jax.lax.broadcasted_iota(jnp.int32, (BLK, N), 1)
    for b in range(NBLK):
        sl = slice(b * BLK, (b + 1) * BLK)
        oh = (iota == jx[sl]).astype(f32)
        gath = _dg(oh, net_b)
        h = jax.nn.relu(_dgT(gath, c2w1[...]) + c2b1[...])
        upd = _dgT(h, c2w2[...]) + c2b2[...]
        out_ref[sl] = net_b[sl] + upd


def _soft_agg(x, idx_col, G, fw, fb, gw, gb, hw, hb):
    fx = _dgT(x, fw) + fb
    gx = _dgT(x, gw) + gb
    gmax = jnp.max(gx, axis=0, keepdims=True)
    ex = jnp.exp(gx - gmax)
    oh = (jax.lax.broadcasted_iota(jnp.int32, (N, G), 1) == idx_col).astype(f32)
    esum = _dgTT(oh, ex)
    ynum = _dgTT(oh, fx * ex)
    y = ynum / jnp.where(esum > 0, esum, 1.0)
    hy = _dgT(y, hw) + hb
    return _dg(oh, hy)


def _gr(x, gw, gb, r1w, r1b, r2w, r2b):
    gate = jax.nn.sigmoid(_dgT(x, gw) + gb)
    res = _dgT(jax.nn.relu(_dgT(x, r1w) + r1b), r2w) + r2b
    return x + gate * res


def _k3(x_ref, kkidx_ref, ijidx_ref, ii_ref,
        akfw, akfb, akgw, akgb, akhw, akhb,
        aifw, aifb, aigw, aigb, aihw, aihb,
        l1g, l1b, g1gw, g1gb, g1r1w, g1r1b, g1r2w, g1r2b,
        l2g, l2b, g2gw, g2gb, g2r1w, g2r1b, g2r2w, g2r2b,
        wdw, bdw, out_net_ref, out_dw_ref):
    x = x_ref[...]
    x = x + _soft_agg(x, kkidx_ref[...], G_KK_C,
                      akfw[...], akfb[...], akgw[...], akgb[...],
                      akhw[...], akhb[...])
    x = x + _soft_agg(x, ijidx_ref[...], G_IJ_C,
                      aifw[...], aifb[...], aigw[...], aigb[...],
                      aihw[...], aihb[...])
    x = _ln(x, l1g[...], l1b[...])
    x = _gr(x, g1gw[...], g1gb[...], g1r1w[...], g1r1b[...],
            g1r2w[...], g1r2b[...])
    x = _ln(x, l2g[...], l2b[...])
    x = _gr(x, g2gw[...], g2gb[...], g2r1w[...], g2r1b[...],
            g2r2w[...], g2r2b[...])
    out_net_ref[...] = x
    r = jax.nn.relu(x)
    dw = _dgT(r, wdw[...]) + bdw[...]
    lane = jax.lax.broadcasted_iota(jnp.int32, (N, 8), 1)
    out_dw_ref[...] = (jnp.where(lane < 2, dw, jax.nn.sigmoid(dw))
                       + ii_ref[...] * 1e-10)


def _sds(shape):
    return jax.ShapeDtypeStruct(shape, f32)


@jax.jit
def _run(net_t, inp_t, corr_t, ii_col, kk_row, kk_col, jj_row, jj_col,
         kkidx_col, ijidx_col, p, wdw, bdw):
    net_a = pl.pallas_call(
        _k1, out_shape=_sds((N, DIM)))(
        corr_t, net_t, inp_t, ii_col,
        p['corr_w1'], p['corr_b1'], p['corr_w2'], p['corr_b2'],
        p['corr_ln_g'], p['corr_ln_b'], p['corr_w3'], p['corr_b3'],
        p['norm_g'], p['norm_b'])

    net_b, jx = pl.pallas_call(
        _k2a, out_shape=[_sds((N, DIM)),
                         jax.ShapeDtypeStruct((N, 1), jnp.int32)])(
        net_a, kk_row, kk_col, jj_row, jj_col,
        p['c1_w1'], p['c1_b1'], p['c1_w2'], p['c1_b2'])

    net_c = pl.pallas_call(
        _k2b, out_shape=_sds((N, DIM)))(
        net_b, jx,
        p['c2_w1'], p['c2_b1'], p['c2_w2'], p['c2_b2'])

    net_f, dw = pl.pallas_call(
        _k3, out_shape=[_sds((N, DIM)), _sds((N, 8))])(
        net_c, kkidx_col, ijidx_col, ii_col,
        p['agg_kk_f_w'], p['agg_kk_f_b'], p['agg_kk_g_w'], p['agg_kk_g_b'],
        p['agg_kk_h_w'], p['agg_kk_h_b'],
        p['agg_ij_f_w'], p['agg_ij_f_b'], p['agg_ij_g_w'], p['agg_ij_g_b'],
        p['agg_ij_h_w'], p['agg_ij_h_b'],
        p['gru_ln1_g'], p['gru_ln1_b'],
        p['gr1_gate_w'], p['gr1_gate_b'], p['gr1_res_w1'], p['gr1_res_b1'],
        p['gr1_res_w2'], p['gr1_res_b2'],
        p['gru_ln2_g'], p['gru_ln2_b'],
        p['gr2_gate_w'], p['gr2_gate_b'], p['gr2_res_w1'], p['gr2_res_b1'],
        p['gr2_res_w2'], p['gr2_res_b2'],
        wdw, bdw)
    return net_f, dw


def kernel(net, inp, corr, flow, ii, jj, kk, kk_idx_map, G_kk, ij_idx_map,
           G_ij, params):
    del flow, G_kk, G_ij
    net_t = jnp.transpose(net[0, :, :, 0], (1, 0))
    inp_t = jnp.transpose(inp[0, :, :, 0], (1, 0))
    corr_t = jnp.transpose(corr[0, :, :, 0], (1, 0))
    ii_col = ii[0].astype(f32)
    jj_col = jj[0].astype(jnp.int32)
    kk_col = kk[0].astype(jnp.int32)
    jj_row = jj_col.reshape(1, N)
    kk_row = kk_col.reshape(1, N)
    kkidx_col = kk_idx_map.astype(jnp.int32).reshape(N, 1)
    ijidx_col = ij_idx_map.astype(jnp.int32).reshape(N, 1)

    p = {k: (v.reshape(1, -1) if v.ndim == 1 else v)
         for k, v in params.items()}
    wdw = jnp.concatenate(
        [params['d_w'], params['w_w'], jnp.zeros((4, DIM), f32)], axis=0)
    bdw = jnp.concatenate(
        [params['d_b'], params['w_b'], jnp.zeros((4,), f32)]).reshape(1, 8)

    net_f, dw = _run(net_t, inp_t, corr_t, ii_col, kk_row, kk_col, jj_row,
                     jj_col, kkidx_col, ijidx_col, p, wdw, bdw)
    return net_f[None], dw[None, :, 0:2], dw[None, :, 2:4]
